# Initial kernel scaffold; baseline (speedup 1.0000x reference)
import jax
import jax.numpy as jnp
from jax.experimental import pallas as pl


def kernel(inputs):
    raise NotImplementedError("write your pallas kernel here")


# speedup vs baseline: 1.8976x; 1.8976x over previous
import jax
import jax.numpy as jnp
from jax.experimental import pallas as pl

_COL_START = 26
_COL_END = 126

_BLOCK_ROWS = 2048


def _slice_kernel(in_ref, out_ref):
    out_ref[...] = in_ref[:, _COL_START:_COL_END]


@jax.jit
def kernel(inputs):
    rows, cols = inputs.shape
    n_out = _COL_END - _COL_START
    grid = (rows // _BLOCK_ROWS,)
    return pl.pallas_call(
        _slice_kernel,
        grid=grid,
        in_specs=[pl.BlockSpec((_BLOCK_ROWS, cols), lambda i: (i, 0))],
        out_specs=pl.BlockSpec((_BLOCK_ROWS, n_out), lambda i: (i, 0)),
        out_shape=jax.ShapeDtypeStruct((rows, n_out), inputs.dtype),
    )(inputs)


# speedup vs baseline: 2.1656x; 1.1412x over previous
import jax
import jax.numpy as jnp
from jax.experimental import pallas as pl

_COL_START = 26
_COL_END = 126

_BLOCK_ROWS = 4096


def _slice_kernel(in_ref, out_ref):
    out_ref[...] = in_ref[:, _COL_START:_COL_END]


@jax.jit
def kernel(inputs):
    rows, cols = inputs.shape
    n_out = _COL_END - _COL_START
    grid = (rows // _BLOCK_ROWS,)
    return pl.pallas_call(
        _slice_kernel,
        grid=grid,
        in_specs=[pl.BlockSpec((_BLOCK_ROWS, cols), lambda i: (i, 0))],
        out_specs=pl.BlockSpec((_BLOCK_ROWS, n_out), lambda i: (i, 0)),
        out_shape=jax.ShapeDtypeStruct((rows, n_out), inputs.dtype),
    )(inputs)


# speedup vs baseline: 2.3427x; 1.0818x over previous
import jax
import jax.numpy as jnp
from jax.experimental import pallas as pl

_COL_START = 26
_COL_END = 126

_BLOCK_ROWS = 8192


def _slice_kernel(in_ref, out_ref):
    out_ref[...] = in_ref[:, _COL_START:_COL_END]


@jax.jit
def kernel(inputs):
    rows, cols = inputs.shape
    n_out = _COL_END - _COL_START
    grid = (rows // _BLOCK_ROWS,)
    return pl.pallas_call(
        _slice_kernel,
        grid=grid,
        in_specs=[pl.BlockSpec((_BLOCK_ROWS, cols), lambda i: (i, 0))],
        out_specs=pl.BlockSpec((_BLOCK_ROWS, n_out), lambda i: (i, 0)),
        out_shape=jax.ShapeDtypeStruct((rows, n_out), inputs.dtype),
    )(inputs)
